# baseline (speedup 1.0000x reference)
import functools

import jax
import jax.numpy as jnp
from jax.experimental import pallas as pl
from jax.experimental.pallas import tpu as pltpu

K = 3
DIM = 3
KFULL = K ** DIM


def _mm_body(a_ref, b_ref, o_ref):
    o_ref[...] = jnp.dot(a_ref[...], b_ref[...],
                         preferred_element_type=jnp.float32)


def _matmul(a, b, bm=400):
    m, kc = a.shape
    nc = b.shape[1]
    assert m % bm == 0
    return pl.pallas_call(
        _mm_body,
        grid=(m // bm,),
        in_specs=[
            pl.BlockSpec((bm, kc), lambda i: (i, 0)),
            pl.BlockSpec((kc, nc), lambda i: (0, 0)),
        ],
        out_specs=pl.BlockSpec((bm, nc), lambda i: (i, 0)),
        out_shape=jax.ShapeDtypeStruct((m, nc), jnp.float32),
    )(a, b)


def _spline_layer(x, W, root, bias, src, dst, frac, i0, deg_inv):
    n = x.shape[0]
    e = src.shape[0]
    out_dim = W.shape[2]
    d_in = x.shape[1]
    wr = jnp.transpose(W, (1, 0, 2)).reshape(d_in, KFULL * out_dim)
    wfull = jnp.concatenate([wr, root], axis=1)
    xw_all = _matmul(x, wfull)
    xw = xw_all[:, :KFULL * out_dim].reshape(n * KFULL, out_dim)
    xroot = xw_all[:, KFULL * out_dim:]
    msg = jnp.zeros((e, out_dim), dtype=jnp.float32)
    for s in range(2 ** DIM):
        b = jnp.ones((e,), dtype=jnp.float32)
        wi = jnp.zeros((e,), dtype=jnp.int32)
        mult = 1
        for d in range(DIM):
            bit = (s >> d) & 1
            b = b * (frac[:, d] if bit else (1.0 - frac[:, d]))
            wi = wi + ((i0[:, d] + bit) % K) * mult
            mult = mult * K
        msg = msg + b[:, None] * xw[src * KFULL + wi]
    agg = jax.ops.segment_sum(msg, dst, num_segments=n)
    return agg * deg_inv[:, None] + xroot + bias


def _bn_elu(h, gamma, beta, eps=1e-5):
    mean = jnp.mean(h, axis=0)
    var = jnp.mean((h - mean) ** 2, axis=0)
    hn = (h - mean) / jnp.sqrt(var + eps) * gamma + beta
    return jnp.where(hn > 0, hn, jnp.expm1(hn))


def kernel(x, edge_attr, pos, W1, root1, b1, g1, be1, W2, root2, b2, g2,
           be2, W3, root3, b3, edge_index):
    n = x.shape[0]
    e = edge_attr.shape[0]
    src = edge_index[0]
    dst = edge_index[1]
    v = edge_attr * K
    iv = jnp.floor(v)
    frac = v - iv
    i0 = iv.astype(jnp.int32) % K
    deg = jax.ops.segment_sum(jnp.ones((e,), jnp.float32), dst,
                              num_segments=n)
    deg_inv = 1.0 / jnp.where(deg > 0, deg, 1.0)

    h = _spline_layer(x, W1, root1, b1, src, dst, frac, i0, deg_inv)
    h = _bn_elu(h, g1, be1)
    h = _spline_layer(h, W2, root2, b2, src, dst, frac, i0, deg_inv)
    h = _bn_elu(h, g2, be2)
    h = jnp.concatenate([h, pos], axis=1)
    hp = jnp.pad(h, ((0, 0), (0, 125)))
    w3p = jnp.pad(jnp.transpose(W3, (1, 0, 2)).reshape(2 * 64 + 3, -1),
                  ((0, 125), (0, 0)))
    root3p = jnp.pad(root3, ((0, 125), (0, 0)))
    W3p = jnp.transpose(w3p.reshape(256, KFULL, -1), (1, 0, 2))
    return _spline_layer(hp, W3p, root3p, b3, src, dst, frac, i0, deg_inv)


# speedup vs baseline: 1.3571x; 1.3571x over previous
import functools

import jax
import jax.numpy as jnp
from jax import lax
from jax.experimental import pallas as pl
from jax.experimental.pallas import tpu as pltpu
from jax.experimental.pallas import tpu_sc as plsc

K = 3
DIM = 3
KFULL = K ** DIM
NTILES = 32
CHUNK = 64
GB = 8


def _mm_body(a_ref, b_ref, o_ref):
    o_ref[...] = jnp.dot(a_ref[...], b_ref[...],
                         preferred_element_type=jnp.float32)


def _matmul(a, b, bm=400):
    m, kc = a.shape
    nc = b.shape[1]
    return pl.pallas_call(
        _mm_body,
        grid=(m // bm,),
        in_specs=[
            pl.BlockSpec((bm, kc), lambda i: (i, 0)),
            pl.BlockSpec((kc, nc), lambda i: (0, 0)),
        ],
        out_specs=pl.BlockSpec((bm, nc), lambda i: (i, 0)),
        out_shape=jax.ShapeDtypeStruct((m, nc), jnp.float32),
    )(a, b)


def _basis_body(ea_ref, ei_ref, wi_ref, b_ref):
    ea = ea_ref[...]
    v = ea * float(K)
    iv = jnp.floor(v)
    frac = v - iv
    i0 = iv.astype(jnp.int32) % K
    src = ei_ref[0:1, :]
    wis = []
    bs = []
    for s in range(2 ** DIM):
        b = None
        wi = None
        mult = 1
        for d in range(DIM):
            bit = (s >> d) & 1
            fd = frac[d:d + 1, :]
            t = fd if bit else 1.0 - fd
            b = t if b is None else b * t
            w = ((i0[d:d + 1, :] + bit) % K) * mult
            wi = w if wi is None else wi + w
            mult *= K
        wis.append(src * KFULL + wi)
        bs.append(b)
    wi_ref[...] = jnp.concatenate(wis, axis=0)
    b_ref[...] = jnp.concatenate(bs, axis=0)


def _basis(edge_attr_t, edge_index, e):
    eb = 3200
    return pl.pallas_call(
        _basis_body,
        grid=(e // eb,),
        in_specs=[
            pl.BlockSpec((DIM, eb), lambda i: (0, i)),
            pl.BlockSpec((2, eb), lambda i: (0, i)),
        ],
        out_specs=[
            pl.BlockSpec((8, eb), lambda i: (0, i)),
            pl.BlockSpec((8, eb), lambda i: (0, i)),
        ],
        out_shape=[
            jax.ShapeDtypeStruct((8, e), jnp.int32),
            jax.ShapeDtypeStruct((8, e), jnp.float32),
        ],
    )(edge_attr_t, edge_index)


GW = 128


def _make_sc_agg(n, out_dim, outp, n_chunks, with_deg):
    per_tile = n_chunks * CHUNK
    irows_per_chunk = CHUNK * 8 // 128
    irows_per_tile = n_chunks * irows_per_chunk
    nrows_tile = (n // 16) // 8 * 8
    nrows_rem = n - 16 * nrows_tile
    kv = out_dim // 16
    n_gb = CHUNK // GB
    mesh = plsc.VectorSubcoreMesh(core_axis_name="c", subcore_axis_name="s")

    @functools.partial(
        pl.kernel,
        out_type=jax.ShapeDtypeStruct((2, n, outp), jnp.float32),
        mesh=mesh,
        scratch_types=[
            pltpu.VMEM_SHARED((n, outp), jnp.float32),
            pltpu.VMEM((4, 8 * GB, GW), jnp.float32),
            pltpu.VMEM((2, irows_per_chunk, 128), jnp.int32),
            pltpu.VMEM((2, irows_per_chunk, 128), jnp.float32),
            pltpu.VMEM((4, CHUNK), jnp.int32),
            pltpu.VMEM((CHUNK, outp), jnp.float32),
            pltpu.SemaphoreType.DMA((4,)),
        ],
    )
    def sc_agg(xw, idx8, bw8, dstp, zeros, out,
               shared, rows_v, idx_v, bw_v, dst_v, msg_v, gsem):
        c = lax.axis_index("c")
        s = lax.axis_index("s")
        wid = s * 2 + c
        irow0 = wid * irows_per_tile
        erow0 = wid * per_tile
        zr0 = s * nrows_tile
        pltpu.sync_copy(zeros.at[pl.ds(zr0, nrows_tile)],
                        shared.at[pl.ds(zr0, nrows_tile)])

        @pl.when(s == 15)
        def _zrem():
            pltpu.sync_copy(zeros.at[pl.ds(16 * nrows_tile, nrows_rem)],
                            shared.at[pl.ds(16 * nrows_tile, nrows_rem)])

        plsc.subcore_barrier()
        pltpu.sync_copy(idx8.at[pl.ds(irow0, irows_per_chunk)],
                        idx_v.at[0])
        pltpu.sync_copy(bw8.at[pl.ds(irow0, irows_per_chunk)],
                        bw_v.at[0])
        pltpu.sync_copy(dstp.at[pl.ds(erow0, CHUNK)], dst_v.at[0])
        for j in range(4):
            pltpu.async_copy(
                xw.at[idx_v.at[0, j // 2, pl.ds((j % 2) * 64, 64)]],
                rows_v.at[j], gsem.at[j])

        def chunk_body(g, carry):
            buf = g % 2
            nbuf = (g + 1) % 2

            @pl.when(g + 1 < n_chunks)
            def _prefetch():
                r1 = irow0 + (g + 1) * irows_per_chunk
                pltpu.sync_copy(idx8.at[pl.ds(r1, irows_per_chunk)],
                                idx_v.at[nbuf])
                pltpu.sync_copy(bw8.at[pl.ds(r1, irows_per_chunk)],
                                bw_v.at[nbuf])
                pltpu.sync_copy(dstp.at[pl.ds(erow0 + (g + 1) * CHUNK,
                                              CHUNK)],
                                dst_v.at[(g + 1) % 4])

            def gb_body(j, cc):
                nb = j % 4
                pltpu.make_async_copy(
                    xw.at[idx_v.at[0, 0, pl.ds(0, 64)]],
                    rows_v.at[nb], gsem.at[nb]).wait()

                def pair_body(p, cc2):
                    boff = (j % 2) * 64 + p * 16
                    bv = bw_v[buf, j // 2, pl.ds(boff, 16)]
                    for half in range(2):
                        rb = p * 16 + half * 8
                        accs = [None] * kv
                        bsum = None
                        for si in range(8):
                            b = bv[half * 8 + si]
                            bsum = b if si == 0 else bsum + b
                            for k in range(kv):
                                r = rows_v[nb, rb + si,
                                           pl.ds(k * 16, 16)]
                                t = r * b
                                accs[k] = t if si == 0 else accs[k] + t
                        eo = j * GB + p * 2 + half
                        for k in range(kv):
                            msg_v[eo, pl.ds(k * 16, 16)] = accs[k]
                        if with_deg:
                            lane = lax.iota(jnp.int32, 16)
                            msg_v[eo, pl.ds(out_dim, 16)] = jnp.where(
                                lane == 0, bsum, 0.0)
                    return cc2

                lax.fori_loop(0, GB // 2, pair_body, 0)
                jp = (j + 4) % n_gb
                gp = g + (j >= 4).astype(jnp.int32)

                @pl.when(gp < n_chunks)
                def _fire():
                    pltpu.async_copy(
                        xw.at[idx_v.at[gp % 2, jp // 2,
                                       pl.ds((jp % 2) * 64, 64)]],
                        rows_v.at[nb], gsem.at[nb])

                return cc

            lax.fori_loop(0, n_gb, gb_body, 0)
            pltpu.sync_copy(msg_v, shared.at[dst_v.at[g % 4]], add=True)
            return carry

        lax.fori_loop(0, n_chunks, chunk_body, 0)
        plsc.subcore_barrier()
        pltpu.sync_copy(shared.at[pl.ds(zr0, nrows_tile)],
                        out.at[c, pl.ds(zr0, nrows_tile)])

        @pl.when(s == 15)
        def _wrem():
            pltpu.sync_copy(shared.at[pl.ds(16 * nrows_tile, nrows_rem)],
                            out.at[c, pl.ds(16 * nrows_tile, nrows_rem)])

    return sc_agg


def _post_body(has_bn, out_dim, a_ref, xr_ref, bias_ref, g_ref, be_ref,
               dinv_ref, o_ref, dinv_out_ref=None):
    aggs = a_ref[0] + a_ref[1]
    if dinv_ref is None:
        deg = aggs[:, out_dim:out_dim + 1]
        deginv = 1.0 / jnp.where(deg > 0, deg, 1.0)
    else:
        deginv = dinv_ref[...]
    h = aggs[:, :out_dim] * deginv + xr_ref[...] + bias_ref[...]
    if has_bn:
        mean = jnp.mean(h, axis=0, keepdims=True)
        var = jnp.mean((h - mean) ** 2, axis=0, keepdims=True)
        hn = (h - mean) / jnp.sqrt(var + 1e-5) * g_ref[...] + be_ref[...]
        o_ref[...] = jnp.where(hn > 0, hn, jnp.exp(jnp.minimum(hn, 0.0)) - 1.0)
    else:
        o_ref[...] = h
    if dinv_out_ref is not None:
        dinv_out_ref[...] = deginv


def _post1(agg2, xroot, bias, gamma, beta, n, out_dim):
    body = lambda a, xr, b, g, be, o, dv: _post_body(
        True, out_dim, a, xr, b, g, be, None, o, dv)
    return pl.pallas_call(
        body,
        out_shape=[
            jax.ShapeDtypeStruct((n, out_dim), jnp.float32),
            jax.ShapeDtypeStruct((n, 1), jnp.float32),
        ],
    )(agg2, xroot, bias.reshape(1, -1), gamma.reshape(1, -1),
      beta.reshape(1, -1))


def _post23(agg2, xroot, bias, gamma, beta, deginv, n, out_dim, has_bn):
    body = lambda a, xr, b, g, be, dv, o: _post_body(
        has_bn, out_dim, a, xr, b, g, be, dv, o)
    return pl.pallas_call(
        body,
        out_shape=jax.ShapeDtypeStruct((n, out_dim), jnp.float32),
    )(agg2, xroot, bias.reshape(1, -1), gamma.reshape(1, -1),
      beta.reshape(1, -1), deginv)


def _layer(x, W, root, bias, idx8, bw8, dstp, zeros, n, n_chunks,
           with_deg):
    d_in = x.shape[1]
    out_dim = W.shape[2]
    outp = GW
    Wg = W if out_dim == GW else jnp.pad(
        W, ((0, 0), (0, 0), (0, GW - out_dim)))
    wr = jnp.transpose(Wg, (1, 0, 2)).reshape(d_in, KFULL * GW)
    wfull = jnp.concatenate([wr, root], axis=1)
    xw_all = _matmul(x, wfull)
    xw = xw_all[:, :KFULL * GW].reshape(n * KFULL, GW)
    xroot = xw_all[:, KFULL * GW:]
    agg2 = _make_sc_agg(n, out_dim, outp, n_chunks, with_deg)(
        xw, idx8, bw8, dstp, zeros)
    return agg2, xroot


def kernel(x, edge_attr, pos, W1, root1, b1, g1, be1, W2, root2, b2, g2,
           be2, W3, root3, b3, edge_index):
    n = x.shape[0]
    e = edge_attr.shape[0]
    per_tile_raw = e // NTILES
    n_chunks = -(-per_tile_raw // CHUNK)
    e_pad = NTILES * n_chunks * CHUNK

    wi8, bw8_raw = _basis(edge_attr.T, edge_index, e)
    idx8 = jnp.pad(wi8.T, ((0, e_pad - e), (0, 0))).reshape(-1, 128)
    bw8 = jnp.pad(bw8_raw.T, ((0, e_pad - e), (0, 0))).reshape(-1, 128)
    dstp = jnp.pad(edge_index[1], (0, e_pad - e))
    zeros128 = jnp.zeros((n, GW), jnp.float32)

    agg2, xroot = _layer(x, W1, root1, b1, idx8, bw8, dstp, zeros128, n,
                         n_chunks, True)
    h, deginv = _post1(agg2, xroot, b1, g1, be1, n, 64)

    agg2, xroot = _layer(h, W2, root2, b2, idx8, bw8, dstp, zeros128, n,
                         n_chunks, False)
    h = _post23(agg2, xroot, b2, g2, be2, deginv, n, 128, True)

    h3 = jnp.concatenate(
        [h, pos, jnp.zeros((n, 125), jnp.float32)], axis=1)
    w3p = jnp.pad(jnp.transpose(W3, (1, 0, 2)).reshape(131, -1),
                  ((0, 125), (0, 0)))
    W3p = jnp.transpose(w3p.reshape(256, KFULL, 128), (1, 0, 2))
    root3p = jnp.pad(root3, ((0, 125), (0, 0)))
    agg2, xroot = _layer(h3, W3p, root3p, b3, idx8, bw8, dstp, zeros128,
                         n, n_chunks, False)
    return _post23(agg2, xroot, b3, b3, b3, deginv, n, 128, False)


# speedup vs baseline: 1.5910x; 1.1723x over previous
import functools

import jax
import jax.numpy as jnp
from jax import lax
from jax.experimental import pallas as pl
from jax.experimental.pallas import tpu as pltpu
from jax.experimental.pallas import tpu_sc as plsc

K = 3
DIM = 3
KFULL = K ** DIM
CHUNK = 64
GB = 8
GW = 128
NC0 = 204
NC1 = 112
EB = 2048


def _table_body(pad_to, x_ref, w_ref, o_ref):
    r = jnp.dot(x_ref[...], w_ref[0], preferred_element_type=jnp.float32)
    if pad_to:
        r = jnp.concatenate(
            [r, jnp.zeros((r.shape[0], pad_to), jnp.float32)], axis=1)
    o_ref[...] = r


def _table_mm(x, W, bm=400):
    n, d_in = x.shape
    out_dim = W.shape[2]
    nb = n // bm
    return pl.pallas_call(
        functools.partial(_table_body, GW - out_dim),
        grid=(nb, KFULL),
        in_specs=[
            pl.BlockSpec((bm, d_in), lambda i, k: (i, 0)),
            pl.BlockSpec((1, d_in, out_dim), lambda i, k: (k, 0, 0)),
        ],
        out_specs=pl.BlockSpec((bm, GW), lambda i, k: (k * nb + i, 0)),
        out_shape=jax.ShapeDtypeStruct((KFULL * n, GW), jnp.float32),
    )(x, W)


def _basis_body(n, e_real, ea_ref, ei_ref, wi_ref, b_ref):
    i = pl.program_id(0)
    ea = ea_ref[...]
    v = ea * float(K)
    iv = jnp.floor(v)
    frac = v - iv
    i0 = iv.astype(jnp.int32) % K
    src = ei_ref[0:1, :]
    wis = []
    bs = []
    for s in range(2 ** DIM):
        b = None
        wi = None
        mult = 1
        for d in range(DIM):
            bit = (s >> d) & 1
            fd = frac[d:d + 1, :]
            t = fd if bit else 1.0 - fd
            b = t if b is None else b * t
            w = ((i0[d:d + 1, :] + bit) % K) * mult
            wi = w if wi is None else wi + w
            mult *= K
        wis.append(wi * n + src)
        bs.append(b)
    edge = i * EB + lax.broadcasted_iota(jnp.int32, (1, EB), 1)
    mask = edge < e_real
    wi_ref[...] = jnp.concatenate(wis, axis=0)
    b_ref[...] = jnp.where(mask, jnp.concatenate(bs, axis=0), 0.0)


def _basis(edge_attr_t, edge_index, n, e_real, e_pad):
    nb = e_pad // EB
    return pl.pallas_call(
        functools.partial(_basis_body, n, e_real),
        grid=(nb,),
        in_specs=[
            pl.BlockSpec((DIM, EB), lambda i: (0, i)),
            pl.BlockSpec((2, EB), lambda i: (0, i)),
        ],
        out_specs=[
            pl.BlockSpec((8, EB), lambda i: (0, i)),
            pl.BlockSpec((8, EB), lambda i: (0, i)),
        ],
        out_shape=[
            jax.ShapeDtypeStruct((8, e_pad), jnp.int32),
            jax.ShapeDtypeStruct((8, e_pad), jnp.float32),
        ],
    )(edge_attr_t, edge_index)


def _make_sc_agg(n, out_dim, with_deg):
    irows_per_chunk = CHUNK * 8 // 128
    nrows_tile = (n // 16) // 8 * 8
    nrows_rem = n - 16 * nrows_tile
    kv = out_dim // 16
    n_gb = CHUNK // GB
    mesh = plsc.VectorSubcoreMesh(core_axis_name="c", subcore_axis_name="s")

    @functools.partial(
        pl.kernel,
        out_type=jax.ShapeDtypeStruct((2, n, GW), jnp.float32),
        mesh=mesh,
        scratch_types=[
            pltpu.VMEM_SHARED((n, GW), jnp.float32),
            pltpu.VMEM((4, 8 * GB, GW), jnp.float32),
            pltpu.VMEM((2, irows_per_chunk, 128), jnp.int32),
            pltpu.VMEM((2, irows_per_chunk, 128), jnp.float32),
            pltpu.VMEM((4, CHUNK), jnp.int32),
            pltpu.VMEM((CHUNK, GW), jnp.float32),
            pltpu.SemaphoreType.DMA((4,)),
        ],
    )
    def sc_agg(xw, idx8, bw8, dstp, zeros, out,
               shared, rows_v, idx_v, bw_v, dst_v, msg_v, gsem):
        c = lax.axis_index("c")
        s = lax.axis_index("s")
        chunk_base = s * (NC0 + NC1) + c * NC0
        nc = jnp.where(c == 0, NC0, NC1)
        irow0 = chunk_base * irows_per_chunk
        erow0 = chunk_base * CHUNK
        zr0 = s * nrows_tile
        pltpu.sync_copy(zeros.at[pl.ds(zr0, nrows_tile)],
                        shared.at[pl.ds(zr0, nrows_tile)])

        @pl.when(s == 15)
        def _zrem():
            pltpu.sync_copy(zeros.at[pl.ds(16 * nrows_tile, nrows_rem)],
                            shared.at[pl.ds(16 * nrows_tile, nrows_rem)])

        plsc.subcore_barrier()
        pltpu.sync_copy(idx8.at[pl.ds(irow0, irows_per_chunk)],
                        idx_v.at[0])
        pltpu.sync_copy(bw8.at[pl.ds(irow0, irows_per_chunk)],
                        bw_v.at[0])
        pltpu.sync_copy(dstp.at[pl.ds(erow0, CHUNK)], dst_v.at[0])
        for j in range(4):
            pltpu.async_copy(
                xw.at[idx_v.at[0, j // 2, pl.ds((j % 2) * 64, 64)]],
                rows_v.at[j], gsem.at[j])

        def chunk_body(g, carry):
            buf = g % 2
            nbuf = (g + 1) % 2

            @pl.when(g + 1 < nc)
            def _prefetch():
                r1 = irow0 + (g + 1) * irows_per_chunk
                pltpu.sync_copy(idx8.at[pl.ds(r1, irows_per_chunk)],
                                idx_v.at[nbuf])
                pltpu.sync_copy(bw8.at[pl.ds(r1, irows_per_chunk)],
                                bw_v.at[nbuf])
                pltpu.sync_copy(dstp.at[pl.ds(erow0 + (g + 1) * CHUNK,
                                              CHUNK)],
                                dst_v.at[(g + 1) % 4])

            def gb_body(j, cc):
                nb = j % 4
                pltpu.make_async_copy(
                    xw.at[idx_v.at[0, 0, pl.ds(0, 64)]],
                    rows_v.at[nb], gsem.at[nb]).wait()

                def pair_body(p, cc2):
                    boff = (j % 2) * 64 + p * 16
                    bv = bw_v[buf, j // 2, pl.ds(boff, 16)]
                    for half in range(2):
                        rb = p * 16 + half * 8
                        accs = [None] * kv
                        bsum = None
                        for si in range(8):
                            b = bv[half * 8 + si]
                            bsum = b if si == 0 else bsum + b
                            for k in range(kv):
                                r = rows_v[nb, rb + si,
                                           pl.ds(k * 16, 16)]
                                t = r * b
                                accs[k] = t if si == 0 else accs[k] + t
                        eo = j * GB + p * 2 + half
                        for k in range(kv):
                            msg_v[eo, pl.ds(k * 16, 16)] = accs[k]
                        if with_deg:
                            lane = lax.iota(jnp.int32, 16)
                            msg_v[eo, pl.ds(out_dim, 16)] = jnp.where(
                                lane == 0, bsum, 0.0)
                    return cc2

                lax.fori_loop(0, GB // 2, pair_body, 0)
                jp = (j + 4) % n_gb
                gp = g + (j >= 4).astype(jnp.int32)

                @pl.when(gp < nc)
                def _fire():
                    pltpu.async_copy(
                        xw.at[idx_v.at[gp % 2, jp // 2,
                                       pl.ds((jp % 2) * 64, 64)]],
                        rows_v.at[nb], gsem.at[nb])

                return cc

            lax.fori_loop(0, n_gb, gb_body, 0)
            pltpu.sync_copy(msg_v, shared.at[dst_v.at[g % 4]], add=True)
            return carry

        lax.fori_loop(0, nc, chunk_body, 0)
        plsc.subcore_barrier()
        pltpu.sync_copy(shared.at[pl.ds(zr0, nrows_tile)],
                        out.at[c, pl.ds(zr0, nrows_tile)])

        @pl.when(s == 15)
        def _wrem():
            pltpu.sync_copy(shared.at[pl.ds(16 * nrows_tile, nrows_rem)],
                            out.at[c, pl.ds(16 * nrows_tile, nrows_rem)])

    return sc_agg


def _bn_elu(h, g, be):
    mean = jnp.mean(h, axis=0, keepdims=True)
    var = jnp.mean((h - mean) ** 2, axis=0, keepdims=True)
    hn = (h - mean) / jnp.sqrt(var + 1e-5) * g + be
    return jnp.where(hn > 0, hn, jnp.exp(jnp.minimum(hn, 0.0)) - 1.0)


def _post1_body(out_dim, a_ref, x_ref, rt_ref, b_ref, g_ref, be_ref,
                o_ref, dinv_ref):
    aggs = a_ref[0] + a_ref[1]
    deg = aggs[:, out_dim:out_dim + 1]
    deginv = 1.0 / jnp.where(deg > 0, deg, 1.0)
    xroot = jnp.dot(x_ref[...], rt_ref[...],
                    preferred_element_type=jnp.float32)
    h = aggs[:, :out_dim] * deginv + xroot + b_ref[...]
    o_ref[...] = _bn_elu(h, g_ref[...], be_ref[...])
    dinv_ref[...] = deginv


def _post2_body(out_dim, a_ref, x_ref, rt_ref, b_ref, g_ref, be_ref,
                dv_ref, pos_ref, o_ref):
    aggs = a_ref[0] + a_ref[1]
    xroot = jnp.dot(x_ref[...], rt_ref[...],
                    preferred_element_type=jnp.float32)
    h = aggs[:, :out_dim] * dv_ref[...] + xroot + b_ref[...]
    act = _bn_elu(h, g_ref[...], be_ref[...])
    o_ref[...] = jnp.concatenate([act, pos_ref[...]], axis=1)


def _post3_body(out_dim, a_ref, x_ref, rt_ref, b_ref, dv_ref, o_ref):
    aggs = a_ref[0] + a_ref[1]
    xroot = jnp.dot(x_ref[...], rt_ref[...],
                    preferred_element_type=jnp.float32)
    o_ref[...] = aggs[:, :out_dim] * dv_ref[...] + xroot + b_ref[...]


def kernel(x, edge_attr, pos, W1, root1, b1, g1, be1, W2, root2, b2, g2,
           be2, W3, root3, b3, edge_index):
    n = x.shape[0]
    e = edge_attr.shape[0]
    e_pad = 16 * (NC0 + NC1) * CHUNK
    assert e_pad >= e

    ea_t = jnp.pad(edge_attr.T, ((0, 0), (0, e_pad - e)))
    ei_p = jnp.pad(edge_index, ((0, 0), (0, e_pad - e)))
    wi8, b8 = _basis(ea_t, ei_p, n, e, e_pad)
    idx8 = wi8.T.reshape(e_pad // 16, 128)
    bw8 = b8.T.reshape(e_pad // 16, 128)
    dstp = ei_p[1]
    zeros = jnp.zeros((n, GW), jnp.float32)

    xw = _table_mm(x, W1)
    agg2 = _make_sc_agg(n, 64, True)(xw, idx8, bw8, dstp, zeros)
    h, deginv = pl.pallas_call(
        functools.partial(_post1_body, 64),
        out_shape=[
            jax.ShapeDtypeStruct((n, 64), jnp.float32),
            jax.ShapeDtypeStruct((n, 1), jnp.float32),
        ],
    )(agg2, x, root1, b1.reshape(1, -1), g1.reshape(1, -1),
      be1.reshape(1, -1))

    xw = _table_mm(h, W2)
    agg2 = _make_sc_agg(n, 128, False)(xw, idx8, bw8, dstp, zeros)
    h3 = pl.pallas_call(
        functools.partial(_post2_body, 128),
        out_shape=jax.ShapeDtypeStruct((n, 131), jnp.float32),
    )(agg2, h, root2, b2.reshape(1, -1), g2.reshape(1, -1),
      be2.reshape(1, -1), deginv, pos)

    xw = _table_mm(h3, W3)
    agg2 = _make_sc_agg(n, 128, False)(xw, idx8, bw8, dstp, zeros)
    return pl.pallas_call(
        functools.partial(_post3_body, 128),
        out_shape=jax.ShapeDtypeStruct((n, 128), jnp.float32),
    )(agg2, h3, root3, b3.reshape(1, -1), deginv)


# speedup vs baseline: 2.2050x; 1.3859x over previous
import functools

import jax
import jax.numpy as jnp
from jax import lax
from jax.experimental import pallas as pl
from jax.experimental.pallas import tpu as pltpu
from jax.experimental.pallas import tpu_sc as plsc

K = 3
DIM = 3
KFULL = K ** DIM
CHUNK = 64
GB = 8
GW = 128
NC0 = 172
NC1 = 144
EB = 2048


def _table_body(pad_to, x_ref, w_ref, o_ref):
    r = jnp.dot(x_ref[...], w_ref[0], preferred_element_type=jnp.float32)
    if pad_to:
        r = jnp.concatenate(
            [r, jnp.zeros((r.shape[0], pad_to), jnp.float32)], axis=1)
    o_ref[...] = r


def _table_mm(x, W, bm=2000):
    n, d_in = x.shape
    out_dim = W.shape[2]
    nb = n // bm
    return pl.pallas_call(
        functools.partial(_table_body, GW - out_dim),
        grid=(nb, KFULL),
        in_specs=[
            pl.BlockSpec((bm, d_in), lambda i, k: (i, 0)),
            pl.BlockSpec((1, d_in, out_dim), lambda i, k: (k, 0, 0)),
        ],
        out_specs=pl.BlockSpec((bm, GW), lambda i, k: (k * nb + i, 0)),
        out_shape=jax.ShapeDtypeStruct((KFULL * n, GW), jnp.float32),
    )(x, W)


def _basis_body(n, e_real, ea_ref, ei_ref, wi_ref, b_ref):
    i = pl.program_id(0)
    ea = ea_ref[...]
    v = ea * float(K)
    iv = jnp.floor(v)
    frac = v - iv
    i0 = iv.astype(jnp.int32) % K
    src = ei_ref[0:1, :]
    wis = []
    bs = []
    for s in range(2 ** DIM):
        b = None
        wi = None
        mult = 1
        for d in range(DIM):
            bit = (s >> d) & 1
            fd = frac[d:d + 1, :]
            t = fd if bit else 1.0 - fd
            b = t if b is None else b * t
            w = ((i0[d:d + 1, :] + bit) % K) * mult
            wi = w if wi is None else wi + w
            mult *= K
        wis.append(wi * n + src)
        bs.append(b)
    edge = i * EB + lax.broadcasted_iota(jnp.int32, (1, EB), 1)
    mask = edge < e_real
    wi_ref[...] = jnp.concatenate(wis, axis=0)
    b_ref[...] = jnp.where(mask, jnp.concatenate(bs, axis=0), 0.0)


def _basis(edge_attr_t, edge_index, n, e_real, e_pad):
    nb = e_pad // EB
    return pl.pallas_call(
        functools.partial(_basis_body, n, e_real),
        grid=(nb,),
        in_specs=[
            pl.BlockSpec((DIM, EB), lambda i: (0, i)),
            pl.BlockSpec((2, EB), lambda i: (0, i)),
        ],
        out_specs=[
            pl.BlockSpec((8, EB), lambda i: (0, i)),
            pl.BlockSpec((8, EB), lambda i: (0, i)),
        ],
        out_shape=[
            jax.ShapeDtypeStruct((8, e_pad), jnp.int32),
            jax.ShapeDtypeStruct((8, e_pad), jnp.float32),
        ],
    )(edge_attr_t, edge_index)


def _make_sc_agg(n, out_dim, with_deg):
    irows_per_chunk = CHUNK * 8 // 128
    nrows_tile = (n // 16) // 8 * 8
    nrows_rem = n - 16 * nrows_tile
    kv = out_dim // 16
    n_gb = CHUNK // GB
    mesh = plsc.VectorSubcoreMesh(core_axis_name="c", subcore_axis_name="s")

    @functools.partial(
        pl.kernel,
        out_type=jax.ShapeDtypeStruct((2, n, GW), jnp.float32),
        mesh=mesh,
        scratch_types=[
            pltpu.VMEM_SHARED((n, GW), jnp.float32),
            pltpu.VMEM((4, 8 * GB, GW), jnp.float32),
            pltpu.VMEM((2, irows_per_chunk, 128), jnp.int32),
            pltpu.VMEM((2, irows_per_chunk, 128), jnp.float32),
            pltpu.VMEM((4, CHUNK), jnp.int32),
            pltpu.VMEM((CHUNK, GW), jnp.float32),
            pltpu.SemaphoreType.DMA((4,)),
        ],
    )
    def sc_agg(xw, idx8, bw8, dstp, zeros, out,
               shared, rows_v, idx_v, bw_v, dst_v, msg_v, gsem):
        c = lax.axis_index("c")
        s = lax.axis_index("s")
        chunk_base = s * (NC0 + NC1) + c * NC0
        nc = jnp.where(c == 0, NC0, NC1)
        irow0 = chunk_base * irows_per_chunk
        erow0 = chunk_base * CHUNK
        zr0 = s * nrows_tile
        pltpu.sync_copy(zeros.at[pl.ds(zr0, nrows_tile)],
                        shared.at[pl.ds(zr0, nrows_tile)])

        @pl.when(s == 15)
        def _zrem():
            pltpu.sync_copy(zeros.at[pl.ds(16 * nrows_tile, nrows_rem)],
                            shared.at[pl.ds(16 * nrows_tile, nrows_rem)])

        plsc.subcore_barrier()
        pltpu.sync_copy(idx8.at[pl.ds(irow0, irows_per_chunk)],
                        idx_v.at[0])
        pltpu.sync_copy(bw8.at[pl.ds(irow0, irows_per_chunk)],
                        bw_v.at[0])
        pltpu.sync_copy(dstp.at[pl.ds(erow0, CHUNK)], dst_v.at[0])
        for j in range(4):
            pltpu.async_copy(
                xw.at[idx_v.at[0, j // 2, pl.ds((j % 2) * 64, 64)]],
                rows_v.at[j], gsem.at[j])

        def chunk_body(g, carry):
            buf = g % 2
            nbuf = (g + 1) % 2

            @pl.when(g + 1 < nc)
            def _prefetch():
                r1 = irow0 + (g + 1) * irows_per_chunk
                pltpu.sync_copy(idx8.at[pl.ds(r1, irows_per_chunk)],
                                idx_v.at[nbuf])
                pltpu.sync_copy(bw8.at[pl.ds(r1, irows_per_chunk)],
                                bw_v.at[nbuf])
                pltpu.sync_copy(dstp.at[pl.ds(erow0 + (g + 1) * CHUNK,
                                              CHUNK)],
                                dst_v.at[(g + 1) % 4])

            def gb_body(j, cc):
                nb = j % 4
                pltpu.make_async_copy(
                    xw.at[idx_v.at[0, 0, pl.ds(0, 64)]],
                    rows_v.at[nb], gsem.at[nb]).wait()

                def pair_body(p, cc2):
                    boff = (j % 2) * 64 + p * 16
                    bv = bw_v[buf, j // 2, pl.ds(boff, 16)]
                    for half in range(2):
                        rb = p * 16 + half * 8
                        accs = [None] * kv
                        bsum = None
                        for si in range(8):
                            b = bv[half * 8 + si]
                            bsum = b if si == 0 else bsum + b
                            for k in range(kv):
                                r = rows_v[nb, rb + si,
                                           pl.ds(k * 16, 16)]
                                t = r * b
                                accs[k] = t if si == 0 else accs[k] + t
                        eo = j * GB + p * 2 + half
                        for k in range(kv):
                            msg_v[eo, pl.ds(k * 16, 16)] = accs[k]
                        if with_deg:
                            lane = lax.iota(jnp.int32, 16)
                            msg_v[eo, pl.ds(out_dim, 16)] = jnp.where(
                                lane == 0, bsum, 0.0)
                    return cc2

                lax.fori_loop(0, GB // 2, pair_body, 0)
                jp = (j + 4) % n_gb
                gp = g + (j >= 4).astype(jnp.int32)

                @pl.when(gp < nc)
                def _fire():
                    pltpu.async_copy(
                        xw.at[idx_v.at[gp % 2, jp // 2,
                                       pl.ds((jp % 2) * 64, 64)]],
                        rows_v.at[nb], gsem.at[nb])

                return cc

            lax.fori_loop(0, n_gb, gb_body, 0)
            pltpu.sync_copy(msg_v, shared.at[dst_v.at[g % 4]], add=True)
            return carry

        lax.fori_loop(0, nc, chunk_body, 0)
        plsc.subcore_barrier()
        pltpu.sync_copy(shared.at[pl.ds(zr0, nrows_tile)],
                        out.at[c, pl.ds(zr0, nrows_tile)])

        @pl.when(s == 15)
        def _wrem():
            pltpu.sync_copy(shared.at[pl.ds(16 * nrows_tile, nrows_rem)],
                            out.at[c, pl.ds(16 * nrows_tile, nrows_rem)])

    return sc_agg


def _bn_elu(h, g, be):
    mean = jnp.mean(h, axis=0, keepdims=True)
    var = jnp.mean((h - mean) ** 2, axis=0, keepdims=True)
    hn = (h - mean) / jnp.sqrt(var + 1e-5) * g + be
    return jnp.where(hn > 0, hn, jnp.exp(jnp.minimum(hn, 0.0)) - 1.0)


def _post1_body(out_dim, a_ref, x_ref, rt_ref, b_ref, g_ref, be_ref,
                o_ref, dinv_ref):
    aggs = a_ref[0] + a_ref[1]
    deg = aggs[:, out_dim:out_dim + 1]
    deginv = 1.0 / jnp.where(deg > 0, deg, 1.0)
    xroot = jnp.dot(x_ref[...], rt_ref[...],
                    preferred_element_type=jnp.float32)
    h = aggs[:, :out_dim] * deginv + xroot + b_ref[...]
    o_ref[...] = _bn_elu(h, g_ref[...], be_ref[...])
    dinv_ref[...] = deginv


def _post2_body(out_dim, a_ref, x_ref, rt_ref, b_ref, g_ref, be_ref,
                dv_ref, pos_ref, o_ref):
    aggs = a_ref[0] + a_ref[1]
    xroot = jnp.dot(x_ref[...], rt_ref[...],
                    preferred_element_type=jnp.float32)
    h = aggs[:, :out_dim] * dv_ref[...] + xroot + b_ref[...]
    act = _bn_elu(h, g_ref[...], be_ref[...])
    o_ref[...] = jnp.concatenate([act, pos_ref[...]], axis=1)


def _post3_body(out_dim, a_ref, x_ref, rt_ref, b_ref, dv_ref, o_ref):
    aggs = a_ref[0] + a_ref[1]
    xroot = jnp.dot(x_ref[...], rt_ref[...],
                    preferred_element_type=jnp.float32)
    o_ref[...] = aggs[:, :out_dim] * dv_ref[...] + xroot + b_ref[...]


def kernel(x, edge_attr, pos, W1, root1, b1, g1, be1, W2, root2, b2, g2,
           be2, W3, root3, b3, edge_index):
    n = x.shape[0]
    e = edge_attr.shape[0]
    e_pad = 16 * (NC0 + NC1) * CHUNK
    assert e_pad >= e

    ea_t = jnp.pad(edge_attr.T, ((0, 0), (0, e_pad - e)))
    ei_p = jnp.pad(edge_index, ((0, 0), (0, e_pad - e)))
    wi8, b8 = _basis(ea_t, ei_p, n, e, e_pad)
    idx8 = wi8.T.reshape(e_pad // 16, 128)
    bw8 = b8.T.reshape(e_pad // 16, 128)
    dstp = ei_p[1]
    zeros = jnp.zeros((n, GW), jnp.float32)

    xw = _table_mm(x, W1)
    agg2 = _make_sc_agg(n, 64, True)(xw, idx8, bw8, dstp, zeros)
    h, deginv = pl.pallas_call(
        functools.partial(_post1_body, 64),
        out_shape=[
            jax.ShapeDtypeStruct((n, 64), jnp.float32),
            jax.ShapeDtypeStruct((n, 1), jnp.float32),
        ],
    )(agg2, x, root1, b1.reshape(1, -1), g1.reshape(1, -1),
      be1.reshape(1, -1))

    xw = _table_mm(h, W2)
    agg2 = _make_sc_agg(n, 128, False)(xw, idx8, bw8, dstp, zeros)
    h3 = pl.pallas_call(
        functools.partial(_post2_body, 128),
        out_shape=jax.ShapeDtypeStruct((n, 131), jnp.float32),
    )(agg2, h, root2, b2.reshape(1, -1), g2.reshape(1, -1),
      be2.reshape(1, -1), deginv, pos)

    xw = _table_mm(h3, W3)
    agg2 = _make_sc_agg(n, 128, False)(xw, idx8, bw8, dstp, zeros)
    return pl.pallas_call(
        functools.partial(_post3_body, 128),
        out_shape=jax.ShapeDtypeStruct((n, 128), jnp.float32),
    )(agg2, h3, root3, b3.reshape(1, -1), deginv)


# speedup vs baseline: 2.5294x; 1.1471x over previous
import functools

import jax
import jax.numpy as jnp
from jax import lax
from jax.experimental import pallas as pl
from jax.experimental.pallas import tpu as pltpu
from jax.experimental.pallas import tpu_sc as plsc

K = 3
DIM = 3
KFULL = K ** DIM
CHUNK = 48
GB = 8
GW = 128
NC0 = 232
NC1 = 192
EB = 2048


def _table_body(pad_to, x_ref, w_ref, o_ref):
    r = jnp.dot(x_ref[...], w_ref[0], preferred_element_type=jnp.float32)
    if pad_to:
        r = jnp.concatenate(
            [r, jnp.zeros((r.shape[0], pad_to), jnp.float32)], axis=1)
    o_ref[...] = r


def _table_mm(x, W, bm=2000):
    n, d_in = x.shape
    out_dim = W.shape[2]
    nb = n // bm
    return pl.pallas_call(
        functools.partial(_table_body, GW - out_dim),
        grid=(nb, KFULL),
        in_specs=[
            pl.BlockSpec((bm, d_in), lambda i, k: (i, 0)),
            pl.BlockSpec((1, d_in, out_dim), lambda i, k: (k, 0, 0)),
        ],
        out_specs=pl.BlockSpec((bm, GW), lambda i, k: (k * nb + i, 0)),
        out_shape=jax.ShapeDtypeStruct((KFULL * n, GW), jnp.float32),
    )(x, W)


def _basis_body(n, e_real, ea_ref, ei_ref, wi_ref, b_ref):
    i = pl.program_id(0)
    ea = ea_ref[...]
    v = ea * float(K)
    iv = jnp.floor(v)
    frac = v - iv
    i0 = iv.astype(jnp.int32) % K
    src = ei_ref[0:1, :]
    wis = []
    bs = []
    for s in range(2 ** DIM):
        b = None
        wi = None
        mult = 1
        for d in range(DIM):
            bit = (s >> d) & 1
            fd = frac[d:d + 1, :]
            t = fd if bit else 1.0 - fd
            b = t if b is None else b * t
            w = ((i0[d:d + 1, :] + bit) % K) * mult
            wi = w if wi is None else wi + w
            mult *= K
        wis.append(wi * n + src)
        bs.append(b)
    edge = i * EB + lax.broadcasted_iota(jnp.int32, (1, EB), 1)
    mask = edge < e_real
    wi_ref[...] = jnp.concatenate(wis, axis=0)
    b_ref[...] = jnp.where(mask, jnp.concatenate(bs, axis=0), 0.0)


def _basis(edge_attr_t, edge_index, n, e_real, e_pad):
    nb = e_pad // EB
    return pl.pallas_call(
        functools.partial(_basis_body, n, e_real),
        grid=(nb,),
        in_specs=[
            pl.BlockSpec((DIM, EB), lambda i: (0, i)),
            pl.BlockSpec((2, EB), lambda i: (0, i)),
        ],
        out_specs=[
            pl.BlockSpec((8, EB), lambda i: (0, i)),
            pl.BlockSpec((8, EB), lambda i: (0, i)),
        ],
        out_shape=[
            jax.ShapeDtypeStruct((8, e_pad), jnp.int32),
            jax.ShapeDtypeStruct((8, e_pad), jnp.float32),
        ],
    )(edge_attr_t, edge_index)


def _make_sc_agg(n, out_dim, with_deg):
    iw = CHUNK * 8
    nrows_tile = (n // 16) // 8 * 8
    nrows_rem = n - 16 * nrows_tile
    kv = out_dim // 16
    n_gb = CHUNK // GB
    mesh = plsc.VectorSubcoreMesh(core_axis_name="c", subcore_axis_name="s")

    @functools.partial(
        pl.kernel,
        out_type=jax.ShapeDtypeStruct((2, n, GW), jnp.float32),
        mesh=mesh,
        scratch_types=[
            pltpu.VMEM_SHARED((n, GW), jnp.float32),
            pltpu.VMEM((4, 8 * GB, GW), jnp.float32),
            pltpu.VMEM((2, iw), jnp.int32),
            pltpu.VMEM((2, iw), jnp.float32),
            pltpu.VMEM((4, CHUNK), jnp.int32),
            pltpu.VMEM((2, CHUNK, GW), jnp.float32),
            pltpu.SemaphoreType.DMA((4,)),
            pltpu.SemaphoreType.DMA,
            pltpu.SemaphoreType.DMA,
        ],
    )
    def sc_agg(xw, idx8, bw8, dstp, zeros, out,
               shared, rows_v, idx_v, bw_v, dst_v, msg_v, gsem, psem,
               ssem):
        c = lax.axis_index("c")
        s = lax.axis_index("s")
        chunk_base = s * (NC0 + NC1) + c * NC0
        nc = jnp.where(c == 0, NC0, NC1)
        irow0 = chunk_base * iw
        erow0 = chunk_base * CHUNK
        zr0 = s * nrows_tile
        pltpu.sync_copy(zeros.at[pl.ds(zr0, nrows_tile)],
                        shared.at[pl.ds(zr0, nrows_tile)])

        @pl.when(s == 15)
        def _zrem():
            pltpu.sync_copy(zeros.at[pl.ds(16 * nrows_tile, nrows_rem)],
                            shared.at[pl.ds(16 * nrows_tile, nrows_rem)])

        plsc.subcore_barrier()
        pltpu.sync_copy(idx8.at[pl.ds(irow0, iw)], idx_v.at[0])
        pltpu.sync_copy(bw8.at[pl.ds(irow0, iw)], bw_v.at[0])
        pltpu.sync_copy(dstp.at[pl.ds(erow0, CHUNK)], dst_v.at[0])
        for j in range(4):
            pltpu.async_copy(
                xw.at[idx_v.at[0, pl.ds(j * 64, 64)]],
                rows_v.at[j], gsem.at[j])

        def chunk_body(g, carry):
            buf = g % 2
            nbuf = (g + 1) % 2
            mb = g % 2

            @pl.when(g + 1 < nc)
            def _prefetch():
                r1 = irow0 + (g + 1) * iw
                pltpu.async_copy(idx8.at[pl.ds(r1, iw)],
                                 idx_v.at[nbuf], psem)
                pltpu.async_copy(bw8.at[pl.ds(r1, iw)],
                                 bw_v.at[nbuf], psem)
                pltpu.async_copy(dstp.at[pl.ds(erow0 + (g + 1) * CHUNK,
                                               CHUNK)],
                                 dst_v.at[(g + 1) % 4], psem)

            @pl.when(g >= 2)
            def _sdrain():
                pltpu.make_async_copy(
                    msg_v.at[mb], shared.at[dst_v.at[g % 4]],
                    ssem).wait()

            def gb_body(j, cc):
                nb = (2 * g + j) % 4

                @pl.when((j == 2) & (g + 1 < nc))
                def _pdrain():
                    pltpu.make_async_copy(
                        idx8.at[pl.ds(irow0, iw)],
                        idx_v.at[0], psem).wait()
                    pltpu.make_async_copy(
                        bw8.at[pl.ds(irow0, iw)],
                        bw_v.at[0], psem).wait()
                    pltpu.make_async_copy(
                        dstp.at[pl.ds(erow0, CHUNK)], dst_v.at[0],
                        psem).wait()

                pltpu.make_async_copy(
                    xw.at[idx_v.at[0, pl.ds(0, 64)]],
                    rows_v.at[nb], gsem.at[nb]).wait()

                def pair_body(p, cc2):
                    bv = bw_v[buf, pl.ds(j * 64 + p * 16, 16)]
                    for half in range(2):
                        rb = p * 16 + half * 8
                        accs = [None] * kv
                        bsum = None
                        for si in range(8):
                            b = bv[half * 8 + si]
                            bsum = b if si == 0 else bsum + b
                            for k in range(kv):
                                r = rows_v[nb, rb + si,
                                           pl.ds(k * 16, 16)]
                                t = r * b
                                accs[k] = t if si == 0 else accs[k] + t
                        eo = j * GB + p * 2 + half
                        for k in range(kv):
                            msg_v[mb, eo, pl.ds(k * 16, 16)] = accs[k]
                        if with_deg:
                            lane = lax.iota(jnp.int32, 16)
                            msg_v[mb, eo, pl.ds(out_dim, 16)] = \
                                jnp.where(lane == 0, bsum, 0.0)
                    return cc2

                lax.fori_loop(0, GB // 2, pair_body, 0)
                jp = (j + 4) % n_gb
                gp = g + (j >= n_gb - 4).astype(jnp.int32)

                @pl.when(gp < nc)
                def _fire():
                    pltpu.async_copy(
                        xw.at[idx_v.at[gp % 2, pl.ds(jp * 64, 64)]],
                        rows_v.at[nb], gsem.at[nb])

                return cc

            lax.fori_loop(0, n_gb, gb_body, 0)
            pltpu.async_copy(msg_v.at[mb], shared.at[dst_v.at[g % 4]],
                             ssem, add=True)
            return carry

        lax.fori_loop(0, nc, chunk_body, 0)
        for t in range(2):
            pltpu.make_async_copy(msg_v.at[t], shared.at[dst_v.at[0]],
                                  ssem).wait()
        plsc.subcore_barrier()
        pltpu.sync_copy(shared.at[pl.ds(zr0, nrows_tile)],
                        out.at[c, pl.ds(zr0, nrows_tile)])

        @pl.when(s == 15)
        def _wrem():
            pltpu.sync_copy(shared.at[pl.ds(16 * nrows_tile, nrows_rem)],
                            out.at[c, pl.ds(16 * nrows_tile, nrows_rem)])

    return sc_agg


def _bn_elu(h, g, be):
    mean = jnp.mean(h, axis=0, keepdims=True)
    var = jnp.mean((h - mean) ** 2, axis=0, keepdims=True)
    hn = (h - mean) / jnp.sqrt(var + 1e-5) * g + be
    return jnp.where(hn > 0, hn, jnp.exp(jnp.minimum(hn, 0.0)) - 1.0)


def _post1_body(out_dim, a_ref, x_ref, rt_ref, b_ref, g_ref, be_ref,
                o_ref, dinv_ref):
    aggs = a_ref[0] + a_ref[1]
    deg = aggs[:, out_dim:out_dim + 1]
    deginv = 1.0 / jnp.where(deg > 0, deg, 1.0)
    xroot = jnp.dot(x_ref[...], rt_ref[...],
                    preferred_element_type=jnp.float32)
    h = aggs[:, :out_dim] * deginv + xroot + b_ref[...]
    o_ref[...] = _bn_elu(h, g_ref[...], be_ref[...])
    dinv_ref[...] = deginv


def _post2_body(out_dim, a_ref, x_ref, rt_ref, b_ref, g_ref, be_ref,
                dv_ref, pos_ref, o_ref):
    aggs = a_ref[0] + a_ref[1]
    xroot = jnp.dot(x_ref[...], rt_ref[...],
                    preferred_element_type=jnp.float32)
    h = aggs[:, :out_dim] * dv_ref[...] + xroot + b_ref[...]
    act = _bn_elu(h, g_ref[...], be_ref[...])
    o_ref[...] = jnp.concatenate([act, pos_ref[...]], axis=1)


def _post3_body(out_dim, a_ref, x_ref, rt_ref, b_ref, dv_ref, o_ref):
    aggs = a_ref[0] + a_ref[1]
    xroot = jnp.dot(x_ref[...], rt_ref[...],
                    preferred_element_type=jnp.float32)
    o_ref[...] = aggs[:, :out_dim] * dv_ref[...] + xroot + b_ref[...]


def kernel(x, edge_attr, pos, W1, root1, b1, g1, be1, W2, root2, b2, g2,
           be2, W3, root3, b3, edge_index):
    n = x.shape[0]
    e = edge_attr.shape[0]
    e_pad = 16 * (NC0 + NC1) * CHUNK
    assert e_pad >= e

    ea_t = jnp.pad(edge_attr.T, ((0, 0), (0, e_pad - e)))
    ei_p = jnp.pad(edge_index, ((0, 0), (0, e_pad - e)))
    wi8, b8 = _basis(ea_t, ei_p, n, e, e_pad)
    idx8 = wi8.T.reshape(-1)
    bw8 = b8.T.reshape(-1)
    dstp = ei_p[1]
    zeros = jnp.zeros((n, GW), jnp.float32)

    xw = _table_mm(x, W1)
    agg2 = _make_sc_agg(n, 64, True)(xw, idx8, bw8, dstp, zeros)
    h, deginv = pl.pallas_call(
        functools.partial(_post1_body, 64),
        out_shape=[
            jax.ShapeDtypeStruct((n, 64), jnp.float32),
            jax.ShapeDtypeStruct((n, 1), jnp.float32),
        ],
    )(agg2, x, root1, b1.reshape(1, -1), g1.reshape(1, -1),
      be1.reshape(1, -1))

    xw = _table_mm(h, W2)
    agg2 = _make_sc_agg(n, 128, False)(xw, idx8, bw8, dstp, zeros)
    h3 = pl.pallas_call(
        functools.partial(_post2_body, 128),
        out_shape=jax.ShapeDtypeStruct((n, 131), jnp.float32),
    )(agg2, h, root2, b2.reshape(1, -1), g2.reshape(1, -1),
      be2.reshape(1, -1), deginv, pos)

    xw = _table_mm(h3, W3)
    agg2 = _make_sc_agg(n, 128, False)(xw, idx8, bw8, dstp, zeros)
    return pl.pallas_call(
        functools.partial(_post3_body, 128),
        out_shape=jax.ShapeDtypeStruct((n, 128), jnp.float32),
    )(agg2, h3, root3, b3.reshape(1, -1), deginv)


# speedup vs baseline: 2.5583x; 1.0114x over previous
import functools

import jax
import jax.numpy as jnp
from jax import lax
from jax.experimental import pallas as pl
from jax.experimental.pallas import tpu as pltpu
from jax.experimental.pallas import tpu_sc as plsc

K = 3
DIM = 3
KFULL = K ** DIM
CHUNK = 48
GB = 8
GW = 128
NC0 = 248
NC1 = 176
EB = 2048


def _table_body(pad_to, x_ref, w_ref, o_ref):
    r = jnp.dot(x_ref[...], w_ref[0], preferred_element_type=jnp.float32)
    if pad_to:
        r = jnp.concatenate(
            [r, jnp.zeros((r.shape[0], pad_to), jnp.float32)], axis=1)
    o_ref[...] = r


def _table_mm(x, W, bm=2000):
    n, d_in = x.shape
    out_dim = W.shape[2]
    nb = n // bm
    return pl.pallas_call(
        functools.partial(_table_body, GW - out_dim),
        grid=(nb, KFULL),
        in_specs=[
            pl.BlockSpec((bm, d_in), lambda i, k: (i, 0)),
            pl.BlockSpec((1, d_in, out_dim), lambda i, k: (k, 0, 0)),
        ],
        out_specs=pl.BlockSpec((bm, GW), lambda i, k: (k * nb + i, 0)),
        out_shape=jax.ShapeDtypeStruct((KFULL * n, GW), jnp.float32),
    )(x, W)


def _basis_body(n, e_real, ea_ref, ei_ref, wi_ref, b_ref):
    i = pl.program_id(0)
    ea = ea_ref[...]
    v = ea * float(K)
    iv = jnp.floor(v)
    frac = v - iv
    i0 = iv.astype(jnp.int32) % K
    src = ei_ref[0:1, :]
    wis = []
    bs = []
    for s in range(2 ** DIM):
        b = None
        wi = None
        mult = 1
        for d in range(DIM):
            bit = (s >> d) & 1
            fd = frac[d:d + 1, :]
            t = fd if bit else 1.0 - fd
            b = t if b is None else b * t
            w = ((i0[d:d + 1, :] + bit) % K) * mult
            wi = w if wi is None else wi + w
            mult *= K
        wis.append(wi * n + src)
        bs.append(b)
    edge = i * EB + lax.broadcasted_iota(jnp.int32, (1, EB), 1)
    mask = edge < e_real
    wi_ref[...] = jnp.concatenate(wis, axis=0)
    b_ref[...] = jnp.where(mask, jnp.concatenate(bs, axis=0), 0.0)


def _basis(edge_attr_t, edge_index, n, e_real, e_pad):
    nb = e_pad // EB
    return pl.pallas_call(
        functools.partial(_basis_body, n, e_real),
        grid=(nb,),
        in_specs=[
            pl.BlockSpec((DIM, EB), lambda i: (0, i)),
            pl.BlockSpec((2, EB), lambda i: (0, i)),
        ],
        out_specs=[
            pl.BlockSpec((8, EB), lambda i: (0, i)),
            pl.BlockSpec((8, EB), lambda i: (0, i)),
        ],
        out_shape=[
            jax.ShapeDtypeStruct((8, e_pad), jnp.int32),
            jax.ShapeDtypeStruct((8, e_pad), jnp.float32),
        ],
    )(edge_attr_t, edge_index)


def _make_sc_agg(n, out_dim, with_deg):
    iw = CHUNK * 8
    nrows_tile = (n // 16) // 8 * 8
    nrows_rem = n - 16 * nrows_tile
    kv = out_dim // 16
    n_gb = CHUNK // GB
    mesh = plsc.VectorSubcoreMesh(core_axis_name="c", subcore_axis_name="s")

    @functools.partial(
        pl.kernel,
        out_type=jax.ShapeDtypeStruct((2, n, GW), jnp.float32),
        mesh=mesh,
        scratch_types=[
            pltpu.VMEM_SHARED((n, GW), jnp.float32),
            pltpu.VMEM((4, 8 * GB, GW), jnp.float32),
            pltpu.VMEM((2, iw), jnp.int32),
            pltpu.VMEM((2, iw), jnp.float32),
            pltpu.VMEM((4, CHUNK), jnp.int32),
            pltpu.VMEM((2, CHUNK, GW), jnp.float32),
            pltpu.SemaphoreType.DMA((4,)),
            pltpu.SemaphoreType.DMA,
            pltpu.SemaphoreType.DMA,
        ],
    )
    def sc_agg(xw, idx8, bw8, dstp, zeros, out,
               shared, rows_v, idx_v, bw_v, dst_v, msg_v, gsem, psem,
               ssem):
        c = lax.axis_index("c")
        s = lax.axis_index("s")
        chunk_base = s * (NC0 + NC1) + c * NC0
        nc = jnp.where(c == 0, NC0, NC1)
        irow0 = chunk_base * iw
        erow0 = chunk_base * CHUNK
        zr0 = s * nrows_tile
        pltpu.sync_copy(zeros.at[pl.ds(zr0, nrows_tile)],
                        shared.at[pl.ds(zr0, nrows_tile)])

        @pl.when(s == 15)
        def _zrem():
            pltpu.sync_copy(zeros.at[pl.ds(16 * nrows_tile, nrows_rem)],
                            shared.at[pl.ds(16 * nrows_tile, nrows_rem)])

        plsc.subcore_barrier()
        pltpu.sync_copy(idx8.at[pl.ds(irow0, iw)], idx_v.at[0])
        pltpu.sync_copy(bw8.at[pl.ds(irow0, iw)], bw_v.at[0])
        pltpu.sync_copy(dstp.at[pl.ds(erow0, CHUNK)], dst_v.at[0])
        for j in range(4):
            pltpu.async_copy(
                xw.at[idx_v.at[0, pl.ds(j * 64, 64)]],
                rows_v.at[j], gsem.at[j])

        def chunk_body(g, carry):
            buf = g % 2
            nbuf = (g + 1) % 2
            mb = g % 2

            @pl.when(g + 1 < nc)
            def _prefetch():
                r1 = irow0 + (g + 1) * iw
                pltpu.async_copy(idx8.at[pl.ds(r1, iw)],
                                 idx_v.at[nbuf], psem)
                pltpu.async_copy(bw8.at[pl.ds(r1, iw)],
                                 bw_v.at[nbuf], psem)
                pltpu.async_copy(dstp.at[pl.ds(erow0 + (g + 1) * CHUNK,
                                               CHUNK)],
                                 dst_v.at[(g + 1) % 4], psem)

            @pl.when(g >= 2)
            def _sdrain():
                pltpu.make_async_copy(
                    msg_v.at[mb], shared.at[dst_v.at[g % 4]],
                    ssem).wait()

            def gb_body(j, cc):
                nb = (2 * g + j) % 4

                @pl.when((j == 2) & (g + 1 < nc))
                def _pdrain():
                    pltpu.make_async_copy(
                        idx8.at[pl.ds(irow0, iw)],
                        idx_v.at[0], psem).wait()
                    pltpu.make_async_copy(
                        bw8.at[pl.ds(irow0, iw)],
                        bw_v.at[0], psem).wait()
                    pltpu.make_async_copy(
                        dstp.at[pl.ds(erow0, CHUNK)], dst_v.at[0],
                        psem).wait()

                pltpu.make_async_copy(
                    xw.at[idx_v.at[0, pl.ds(0, 64)]],
                    rows_v.at[nb], gsem.at[nb]).wait()

                def pair_body(p, cc2):
                    bv = bw_v[buf, pl.ds(j * 64 + p * 16, 16)]
                    for half in range(2):
                        rb = p * 16 + half * 8
                        accs = [None] * kv
                        bsum = None
                        for si in range(8):
                            b = bv[half * 8 + si]
                            bsum = b if si == 0 else bsum + b
                            for k in range(kv):
                                r = rows_v[nb, rb + si,
                                           pl.ds(k * 16, 16)]
                                t = r * b
                                accs[k] = t if si == 0 else accs[k] + t
                        eo = j * GB + p * 2 + half
                        for k in range(kv):
                            msg_v[mb, eo, pl.ds(k * 16, 16)] = accs[k]
                        if with_deg:
                            lane = lax.iota(jnp.int32, 16)
                            msg_v[mb, eo, pl.ds(out_dim, 16)] = \
                                jnp.where(lane == 0, bsum, 0.0)
                    return cc2

                lax.fori_loop(0, GB // 2, pair_body, 0)
                jp = (j + 4) % n_gb
                gp = g + (j >= n_gb - 4).astype(jnp.int32)

                @pl.when(gp < nc)
                def _fire():
                    pltpu.async_copy(
                        xw.at[idx_v.at[gp % 2, pl.ds(jp * 64, 64)]],
                        rows_v.at[nb], gsem.at[nb])

                return cc

            lax.fori_loop(0, n_gb, gb_body, 0)
            pltpu.async_copy(msg_v.at[mb], shared.at[dst_v.at[g % 4]],
                             ssem, add=True)
            return carry

        lax.fori_loop(0, nc, chunk_body, 0)
        for t in range(2):
            pltpu.make_async_copy(msg_v.at[t], shared.at[dst_v.at[0]],
                                  ssem).wait()
        plsc.subcore_barrier()
        pltpu.sync_copy(shared.at[pl.ds(zr0, nrows_tile)],
                        out.at[c, pl.ds(zr0, nrows_tile)])

        @pl.when(s == 15)
        def _wrem():
            pltpu.sync_copy(shared.at[pl.ds(16 * nrows_tile, nrows_rem)],
                            out.at[c, pl.ds(16 * nrows_tile, nrows_rem)])

    return sc_agg


def _bn_elu(h, g, be):
    mean = jnp.mean(h, axis=0, keepdims=True)
    var = jnp.mean((h - mean) ** 2, axis=0, keepdims=True)
    hn = (h - mean) / jnp.sqrt(var + 1e-5) * g + be
    return jnp.where(hn > 0, hn, jnp.exp(jnp.minimum(hn, 0.0)) - 1.0)


def _post1_body(out_dim, a_ref, x_ref, rt_ref, b_ref, g_ref, be_ref,
                o_ref, dinv_ref):
    aggs = a_ref[0] + a_ref[1]
    deg = aggs[:, out_dim:out_dim + 1]
    deginv = 1.0 / jnp.where(deg > 0, deg, 1.0)
    xroot = jnp.dot(x_ref[...], rt_ref[...],
                    preferred_element_type=jnp.float32)
    h = aggs[:, :out_dim] * deginv + xroot + b_ref[...]
    o_ref[...] = _bn_elu(h, g_ref[...], be_ref[...])
    dinv_ref[...] = deginv


def _post2_body(out_dim, a_ref, x_ref, rt_ref, b_ref, g_ref, be_ref,
                dv_ref, pos_ref, o_ref):
    aggs = a_ref[0] + a_ref[1]
    xroot = jnp.dot(x_ref[...], rt_ref[...],
                    preferred_element_type=jnp.float32)
    h = aggs[:, :out_dim] * dv_ref[...] + xroot + b_ref[...]
    act = _bn_elu(h, g_ref[...], be_ref[...])
    o_ref[...] = jnp.concatenate([act, pos_ref[...]], axis=1)


def _post3_body(out_dim, a_ref, x_ref, rt_ref, b_ref, dv_ref, o_ref):
    aggs = a_ref[0] + a_ref[1]
    xroot = jnp.dot(x_ref[...], rt_ref[...],
                    preferred_element_type=jnp.float32)
    o_ref[...] = aggs[:, :out_dim] * dv_ref[...] + xroot + b_ref[...]


def kernel(x, edge_attr, pos, W1, root1, b1, g1, be1, W2, root2, b2, g2,
           be2, W3, root3, b3, edge_index):
    n = x.shape[0]
    e = edge_attr.shape[0]
    e_pad = 16 * (NC0 + NC1) * CHUNK
    assert e_pad >= e

    ea_t = jnp.pad(edge_attr.T, ((0, 0), (0, e_pad - e)))
    ei_p = jnp.pad(edge_index, ((0, 0), (0, e_pad - e)))
    wi8, b8 = _basis(ea_t, ei_p, n, e, e_pad)
    idx8 = wi8.T.reshape(-1)
    bw8 = b8.T.reshape(-1)
    dstp = ei_p[1]
    zeros = jnp.zeros((n, GW), jnp.float32)

    xw = _table_mm(x, W1)
    agg2 = _make_sc_agg(n, 64, True)(xw, idx8, bw8, dstp, zeros)
    h, deginv = pl.pallas_call(
        functools.partial(_post1_body, 64),
        out_shape=[
            jax.ShapeDtypeStruct((n, 64), jnp.float32),
            jax.ShapeDtypeStruct((n, 1), jnp.float32),
        ],
    )(agg2, x, root1, b1.reshape(1, -1), g1.reshape(1, -1),
      be1.reshape(1, -1))

    xw = _table_mm(h, W2)
    agg2 = _make_sc_agg(n, 128, False)(xw, idx8, bw8, dstp, zeros)
    h3 = pl.pallas_call(
        functools.partial(_post2_body, 128),
        out_shape=jax.ShapeDtypeStruct((n, 131), jnp.float32),
    )(agg2, h, root2, b2.reshape(1, -1), g2.reshape(1, -1),
      be2.reshape(1, -1), deginv, pos)

    xw = _table_mm(h3, W3)
    agg2 = _make_sc_agg(n, 128, False)(xw, idx8, bw8, dstp, zeros)
    return pl.pallas_call(
        functools.partial(_post3_body, 128),
        out_shape=jax.ShapeDtypeStruct((n, 128), jnp.float32),
    )(agg2, h3, root3, b3.reshape(1, -1), deginv)
